# baseline (speedup 1.0000x reference)
import jax
import jax.numpy as jnp
from jax import lax
from jax.experimental import pallas as pl
from jax.experimental.pallas import tpu as pltpu
from jax.experimental.pallas import tpu_sc as plsc

N = 10000
E = 320000
EL = 100000
D = 128
R_DEC = 2.0
T_DEC = 1.0

NC = 2
NS = 16
NW = NC * NS
CH = 128

NCHE = 2560
E2 = NCHE * CH
KH = NCHE // NW
KAF = 100
KAS = 92
CHS = 80
BA = NS * KAF * CH
KDF = 49
KDS = 0
BD = NS * KDF * CH
EL2 = BD + NS * KDS * CHS

NBIN = N + 8
ROWS_PT = 624

_SC_MESH = plsc.VectorSubcoreMesh(core_axis_name="c", subcore_axis_name="s")


def _row_split(sid, fn):
    @pl.when(sid < NS - 1)
    def _():
        fn(sid * ROWS_PT, ROWS_PT)

    @pl.when(sid == NS - 1)
    def _():
        fn((NS - 1) * ROWS_PT, N - (NS - 1) * ROWS_PT)


def _hist_body(dst_hbm, zeros_hbm, deg0_hbm, deg1_hbm,
               acc, didx, ones_v, sem):
    del sem
    cid = lax.axis_index("c")
    sid = lax.axis_index("s")
    wid = cid * NS + sid

    ones = jnp.full((16,), 1.0, dtype=jnp.float32)
    for j in range(CH // 16):
        ones_v[pl.ds(j * 16, 16)] = ones
    pltpu.sync_copy(dst_hbm.at[pl.ds(wid * KH, KH)], didx)

    @pl.when(sid == 0)
    def _():
        pltpu.sync_copy(zeros_hbm, acc)

    plsc.subcore_barrier()

    def chunk(k, carry):
        pltpu.sync_copy(ones_v, acc.at[didx.at[k]], add=True)
        return carry

    lax.fori_loop(0, KH, chunk, 0)
    plsc.subcore_barrier()

    @pl.when(sid == 0)
    def _():
        @pl.when(cid == 0)
        def _():
            pltpu.sync_copy(acc, deg0_hbm)

        @pl.when(cid == 1)
        def _():
            pltpu.sync_copy(acc, deg1_hbm)


_hist_kernel = pl.kernel(
    _hist_body,
    out_type=(jax.ShapeDtypeStruct((NBIN,), jnp.float32),
              jax.ShapeDtypeStruct((NBIN,), jnp.float32)),
    mesh=_SC_MESH,
    scratch_types=[
        pltpu.VMEM_SHARED((NBIN,), jnp.float32),
        pltpu.VMEM((KH, CH), jnp.int32),
        pltpu.VMEM((CH,), jnp.float32),
        pltpu.SemaphoreType.DMA,
    ],
)


def _agg_body(y_hbm, src_hbm, dst_hbm, zeros_hbm, outa_hbm, outb_hbm,
              acc, s0, s1, d0, d1, s0s, d0s, rows0, rows1, rows_s,
              sem0, sem1, sems0, sems1, semd0, semd1):
    cid = lax.axis_index("c")
    sid = lax.axis_index("s")

    def init(base, cnt):
        @pl.when(cid == 0)
        def _():
            pltpu.sync_copy(y_hbm.at[pl.ds(base, cnt)],
                            acc.at[pl.ds(base, cnt)])

        @pl.when(cid == 1)
        def _():
            pltpu.sync_copy(zeros_hbm.at[pl.ds(base, cnt)],
                            acc.at[pl.ds(base, cnt)])

    _row_split(sid, init)
    plsc.subcore_barrier()

    def run(start, cnt):
        def iload(k, sbuf, dbuf, sems, semd):
            pltpu.async_copy(src_hbm.at[pl.ds((start + k) * CH, CH)],
                             sbuf, sems)
            pltpu.async_copy(dst_hbm.at[pl.ds((start + k) * CH, CH)],
                             dbuf, semd)

        def iwait(k, sbuf, dbuf, sems, semd):
            pltpu.make_async_copy(src_hbm.at[pl.ds((start + k) * CH, CH)],
                                  sbuf, sems).wait()
            pltpu.make_async_copy(dst_hbm.at[pl.ds((start + k) * CH, CH)],
                                  dbuf, semd).wait()

        def gather(sbuf, buf, sem):
            pltpu.async_copy(y_hbm.at[sbuf], buf, sem)

        def gwait(sbuf, buf, sem):
            pltpu.make_async_copy(y_hbm.at[sbuf], buf, sem).wait()

        iload(0, s0, d0, sems0, semd0)
        iwait(0, s0, d0, sems0, semd0)
        gather(s0, rows0, sem0)

        @pl.when(1 < cnt)
        def _():
            iload(1, s1, d1, sems1, semd1)
            iwait(1, s1, d1, sems1, semd1)
            gather(s1, rows1, sem1)

        def pair(j, carry):
            k0 = 2 * j
            gwait(s0, rows0, sem0)
            pltpu.sync_copy(rows0, acc.at[d0], add=True)

            @pl.when(k0 + 2 < cnt)
            def _():
                iload(k0 + 2, s0, d0, sems0, semd0)
                iwait(k0 + 2, s0, d0, sems0, semd0)
                gather(s0, rows0, sem0)

            gwait(s1, rows1, sem1)
            pltpu.sync_copy(rows1, acc.at[d1], add=True)

            @pl.when(k0 + 3 < cnt)
            def _():
                iload(k0 + 3, s1, d1, sems1, semd1)
                iwait(k0 + 3, s1, d1, sems1, semd1)
                gather(s1, rows1, sem1)

            return carry

        lax.fori_loop(0, cnt // 2, pair, 0)
        if cnt % 2:
            gwait(s0, rows0, sem0)
            pltpu.sync_copy(rows0, acc.at[d0], add=True)

    def run_serial(start_edge, cnt):
        def chunk(k, carry):
            off = start_edge + k * CHS
            pltpu.sync_copy(src_hbm.at[pl.ds(off, CHS)], s0s)
            pltpu.sync_copy(dst_hbm.at[pl.ds(off, CHS)], d0s)
            pltpu.async_copy(y_hbm.at[s0s], rows_s, sem0).wait()
            pltpu.sync_copy(rows_s, acc.at[d0s], add=True)
            return carry

        lax.fori_loop(0, cnt, chunk, 0)

    @pl.when(cid == 0)
    def _():
        run(sid * KAF, KAF)

    @pl.when(cid == 1)
    def _():
        run_serial(BA + sid * (KAS * CHS), KAS)

    plsc.subcore_barrier()

    def flush(base, cnt):
        @pl.when(cid == 0)
        def _():
            pltpu.sync_copy(acc.at[pl.ds(base, cnt)],
                            outa_hbm.at[pl.ds(base, cnt)])

        @pl.when(cid == 1)
        def _():
            pltpu.sync_copy(acc.at[pl.ds(base, cnt)],
                            outb_hbm.at[pl.ds(base, cnt)])

    _row_split(sid, flush)


_agg_kernel = pl.kernel(
    _agg_body,
    out_type=(jax.ShapeDtypeStruct((N, D), jnp.float32),
              jax.ShapeDtypeStruct((N, D), jnp.float32)),
    mesh=_SC_MESH,
    scratch_types=[
        pltpu.VMEM_SHARED((NBIN, D), jnp.float32),
        pltpu.VMEM((CH,), jnp.int32),
        pltpu.VMEM((CH,), jnp.int32),
        pltpu.VMEM((CH,), jnp.int32),
        pltpu.VMEM((CH,), jnp.int32),
        pltpu.VMEM((CHS,), jnp.int32),
        pltpu.VMEM((CHS,), jnp.int32),
        pltpu.VMEM((CH, D), jnp.float32),
        pltpu.VMEM((CH, D), jnp.float32),
        pltpu.VMEM((CHS, D), jnp.float32),
        pltpu.SemaphoreType.DMA,
        pltpu.SemaphoreType.DMA,
        pltpu.SemaphoreType.DMA,
        pltpu.SemaphoreType.DMA,
        pltpu.SemaphoreType.DMA,
        pltpu.SemaphoreType.DMA,
    ],
)


def _dec_body(h_hbm, ein_hbm, eout_hbm, embi_hbm, embo_hbm,
              ia0, ia1, ib0, ib1, ias, ibs, ra0, ra1, rb0, rb1, ras, rbs,
              sa0, sa1, sb0, sb1, sia0, sia1, sib0, sib1):
    cid = lax.axis_index("c")
    sid = lax.axis_index("s")

    def run(start, cnt):
        def iload(k, ba, bb, sema, semb):
            pltpu.async_copy(ein_hbm.at[pl.ds((start + k) * CH, CH)],
                             ba, sema)
            pltpu.async_copy(eout_hbm.at[pl.ds((start + k) * CH, CH)],
                             bb, semb)

        def iwait(k, ba, bb, sema, semb):
            pltpu.make_async_copy(ein_hbm.at[pl.ds((start + k) * CH, CH)],
                                  ba, sema).wait()
            pltpu.make_async_copy(eout_hbm.at[pl.ds((start + k) * CH, CH)],
                                  bb, semb).wait()

        def gather(ba, bb, bufa, bufb, sema, semb):
            pltpu.async_copy(h_hbm.at[ba], bufa, sema)
            pltpu.async_copy(h_hbm.at[bb], bufb, semb)

        def gwait(ba, bb, bufa, bufb, sema, semb):
            pltpu.make_async_copy(h_hbm.at[ba], bufa, sema).wait()
            pltpu.make_async_copy(h_hbm.at[bb], bufb, semb).wait()

        def emit(bufa, bufb, k):
            base = (start + k) * CH
            pltpu.sync_copy(bufa, embi_hbm.at[pl.ds(base, CH)])
            pltpu.sync_copy(bufb, embo_hbm.at[pl.ds(base, CH)])

        iload(0, ia0, ib0, sia0, sib0)
        iwait(0, ia0, ib0, sia0, sib0)
        gather(ia0, ib0, ra0, rb0, sa0, sb0)

        @pl.when(1 < cnt)
        def _():
            iload(1, ia1, ib1, sia1, sib1)
            iwait(1, ia1, ib1, sia1, sib1)
            gather(ia1, ib1, ra1, rb1, sa1, sb1)

        def pair(j, carry):
            k0 = 2 * j
            gwait(ia0, ib0, ra0, rb0, sa0, sb0)
            emit(ra0, rb0, k0)

            @pl.when(k0 + 2 < cnt)
            def _():
                iload(k0 + 2, ia0, ib0, sia0, sib0)
                iwait(k0 + 2, ia0, ib0, sia0, sib0)
                gather(ia0, ib0, ra0, rb0, sa0, sb0)

            gwait(ia1, ib1, ra1, rb1, sa1, sb1)
            emit(ra1, rb1, k0 + 1)

            @pl.when(k0 + 3 < cnt)
            def _():
                iload(k0 + 3, ia1, ib1, sia1, sib1)
                iwait(k0 + 3, ia1, ib1, sia1, sib1)
                gather(ia1, ib1, ra1, rb1, sa1, sb1)

            return carry

        lax.fori_loop(0, cnt // 2, pair, 0)
        if cnt % 2:
            gwait(ia0, ib0, ra0, rb0, sa0, sb0)
            emit(ra0, rb0, cnt - 1)

    def run_serial(start_pair, cnt):
        def chunk(k, carry):
            off = start_pair + k * CHS
            pltpu.sync_copy(ein_hbm.at[pl.ds(off, CHS)], ias)
            pltpu.sync_copy(eout_hbm.at[pl.ds(off, CHS)], ibs)
            cpa = pltpu.async_copy(h_hbm.at[ias], ras, sa0)
            cpb = pltpu.async_copy(h_hbm.at[ibs], rbs, sb0)
            cpa.wait()
            cpb.wait()
            pltpu.sync_copy(ras, embi_hbm.at[pl.ds(off, CHS)])
            pltpu.sync_copy(rbs, embo_hbm.at[pl.ds(off, CHS)])
            return carry

        lax.fori_loop(0, cnt, chunk, 0)

    @pl.when(cid == 0)
    def _():
        run(sid * KDF, KDF)

    if KDS:
        @pl.when(cid == 1)
        def _():
            run_serial(BD + sid * (KDS * CHS), KDS)


_dec_kernel = pl.kernel(
    _dec_body,
    out_type=(jax.ShapeDtypeStruct((EL2, D), jnp.float32),
              jax.ShapeDtypeStruct((EL2, D), jnp.float32)),
    mesh=_SC_MESH,
    scratch_types=[
        pltpu.VMEM((CH,), jnp.int32),
        pltpu.VMEM((CH,), jnp.int32),
        pltpu.VMEM((CH,), jnp.int32),
        pltpu.VMEM((CH,), jnp.int32),
        pltpu.VMEM((CHS,), jnp.int32),
        pltpu.VMEM((CHS,), jnp.int32),
        pltpu.VMEM((CH, D), jnp.float32),
        pltpu.VMEM((CH, D), jnp.float32),
        pltpu.VMEM((CH, D), jnp.float32),
        pltpu.VMEM((CH, D), jnp.float32),
        pltpu.VMEM((CHS, D), jnp.float32),
        pltpu.VMEM((CHS, D), jnp.float32),
        pltpu.SemaphoreType.DMA,
        pltpu.SemaphoreType.DMA,
        pltpu.SemaphoreType.DMA,
        pltpu.SemaphoreType.DMA,
        pltpu.SemaphoreType.DMA,
        pltpu.SemaphoreType.DMA,
        pltpu.SemaphoreType.DMA,
        pltpu.SemaphoreType.DMA,
    ],
)


BLK = 1000


def _dinv(da_ref, db_ref):
    return lax.rsqrt(da_ref[...] + db_ref[...] + 1.0)


def _tc1_call(x, W1, dega, degb):
    def body(x_ref, w_ref, da_ref, db_ref, y_ref):
        xw = jnp.dot(x_ref[...], w_ref[...],
                     preferred_element_type=jnp.float32)
        y_ref[...] = xw * _dinv(da_ref, db_ref)

    return pl.pallas_call(
        body,
        grid=(N // BLK,),
        in_specs=[pl.BlockSpec((BLK, D), lambda i: (i, 0)),
                  pl.BlockSpec((D, D), lambda i: (0, 0)),
                  pl.BlockSpec((BLK, 1), lambda i: (i, 0)),
                  pl.BlockSpec((BLK, 1), lambda i: (i, 0))],
        out_specs=pl.BlockSpec((BLK, D), lambda i: (i, 0)),
        out_shape=jax.ShapeDtypeStruct((N, D), jnp.float32),
    )(x, W1, dega, degb)


def _tc2_call(a0, a1, dega, degb, b1, W2):
    def body(a0_ref, a1_ref, da_ref, db_ref, b_ref, w_ref, y_ref):
        dinv = _dinv(da_ref, db_ref)
        h = jnp.maximum(dinv * (a0_ref[...] + a1_ref[...]) + b_ref[...], 0.0)
        y_ref[...] = jnp.dot(h, w_ref[...],
                             preferred_element_type=jnp.float32) * dinv

    return pl.pallas_call(
        body,
        grid=(N // BLK,),
        in_specs=[pl.BlockSpec((BLK, D), lambda i: (i, 0)),
                  pl.BlockSpec((BLK, D), lambda i: (i, 0)),
                  pl.BlockSpec((BLK, 1), lambda i: (i, 0)),
                  pl.BlockSpec((BLK, 1), lambda i: (i, 0)),
                  pl.BlockSpec((D,), lambda i: (0,)),
                  pl.BlockSpec((D, D), lambda i: (0, 0))],
        out_specs=pl.BlockSpec((BLK, D), lambda i: (i, 0)),
        out_shape=jax.ShapeDtypeStruct((N, D), jnp.float32),
    )(a0, a1, dega, degb, b1, W2)


def _tc3_call(c0, c1, dega, degb, b2):
    def body(c0_ref, c1_ref, da_ref, db_ref, b_ref, h_ref):
        h_ref[...] = (_dinv(da_ref, db_ref) * (c0_ref[...] + c1_ref[...])
                      + b_ref[...])

    return pl.pallas_call(
        body,
        grid=(N // BLK,),
        in_specs=[pl.BlockSpec((BLK, D), lambda i: (i, 0)),
                  pl.BlockSpec((BLK, D), lambda i: (i, 0)),
                  pl.BlockSpec((BLK, 1), lambda i: (i, 0)),
                  pl.BlockSpec((BLK, 1), lambda i: (i, 0)),
                  pl.BlockSpec((D,), lambda i: (0,))],
        out_specs=pl.BlockSpec((BLK, D), lambda i: (i, 0)),
        out_shape=jax.ShapeDtypeStruct((N, D), jnp.float32),
    )(c0, c1, dega, degb, b2)


DBLK = 4096


def _tc4_call(embi, embo):
    def body(a_ref, b_ref, p_ref):
        d = a_ref[...] - b_ref[...]
        ones = jnp.ones((D, 1), jnp.float32)
        sq = jnp.dot(d * d, ones, preferred_element_type=jnp.float32)
        p_ref[...] = 1.0 / (jnp.exp((sq + 1e-12 - R_DEC) / T_DEC) + 1.0)

    return pl.pallas_call(
        body,
        grid=((EL + DBLK - 1) // DBLK,),
        in_specs=[pl.BlockSpec((DBLK, D), lambda i: (i, 0)),
                  pl.BlockSpec((DBLK, D), lambda i: (i, 0))],
        out_specs=pl.BlockSpec((DBLK, 1), lambda i: (i, 0)),
        out_shape=jax.ShapeDtypeStruct((EL, 1), jnp.float32),
    )(embi, embo)


def kernel(node_features, edge_index, edge_label_index, W1, b1, W2, b2):
    x = node_features.astype(jnp.float32)
    srcp = jnp.concatenate(
        [edge_index[0], jnp.zeros((E2 - E,), jnp.int32)])
    dstp = jnp.concatenate(
        [edge_index[1], jnp.full((E2 - E,), N, jnp.int32)])
    dstp2 = dstp.reshape(NCHE, CH)
    einp = jnp.concatenate(
        [edge_label_index[0], jnp.zeros((EL2 - EL,), jnp.int32)])
    eoutp = jnp.concatenate(
        [edge_label_index[1], jnp.zeros((EL2 - EL,), jnp.int32)])
    zeros_nd = jnp.zeros((N, D), jnp.float32)
    zeros_n = jnp.zeros((NBIN,), jnp.float32)

    deg0, deg1 = _hist_kernel(dstp2, zeros_n)
    dega = deg0[:N].reshape(N, 1)
    degb = deg1[:N].reshape(N, 1)

    y1 = _tc1_call(x, W1, dega, degb)
    a0, a1 = _agg_kernel(y1, srcp, dstp, zeros_nd)
    y2 = _tc2_call(a0, a1, dega, degb, b1, W2)
    c0, c1 = _agg_kernel(y2, srcp, dstp, zeros_nd)
    h2 = _tc3_call(c0, c1, dega, degb, b2)

    embi, embo = _dec_kernel(h2, einp, eoutp)
    return _tc4_call(embi, embo).reshape(EL)


# speedup vs baseline: 1.2402x; 1.2402x over previous
import jax
import jax.numpy as jnp
from jax import lax
from jax.experimental import pallas as pl
from jax.experimental.pallas import tpu as pltpu
from jax.experimental.pallas import tpu_sc as plsc

N = 10000
E = 320000
EL = 100000
D = 128
R_DEC = 2.0
T_DEC = 1.0

NC = 2
NS = 16
NW = NC * NS
CH = 128

NCHE = 2560
E2 = NCHE * CH
KH = NCHE // NW
KAF = 104
KAS = 84
CHS = 80
BA = NS * KAF * CH
KDF = 49
KDS = 0
BD = NS * KDF * CH
EL2 = BD + NS * KDS * CHS

NBIN = N + 8
ROWS_PT = 624

_SC_MESH = plsc.VectorSubcoreMesh(core_axis_name="c", subcore_axis_name="s")


def _row_split(sid, fn):
    @pl.when(sid < NS - 1)
    def _():
        fn(sid * ROWS_PT, ROWS_PT)

    @pl.when(sid == NS - 1)
    def _():
        fn((NS - 1) * ROWS_PT, N - (NS - 1) * ROWS_PT)


def _hist_body(dst_hbm, zeros_hbm, deg0_hbm, deg1_hbm,
               acc, didx, ones_v, sem):
    del sem
    cid = lax.axis_index("c")
    sid = lax.axis_index("s")
    wid = cid * NS + sid

    ones = jnp.full((16,), 1.0, dtype=jnp.float32)
    for j in range(CH // 16):
        ones_v[pl.ds(j * 16, 16)] = ones
    pltpu.sync_copy(dst_hbm.at[pl.ds(wid * KH, KH)], didx)

    @pl.when(sid == 0)
    def _():
        pltpu.sync_copy(zeros_hbm, acc)

    plsc.subcore_barrier()

    def chunk(k, carry):
        pltpu.sync_copy(ones_v, acc.at[didx.at[k]], add=True)
        return carry

    lax.fori_loop(0, KH, chunk, 0)
    plsc.subcore_barrier()

    @pl.when(sid == 0)
    def _():
        @pl.when(cid == 0)
        def _():
            pltpu.sync_copy(acc, deg0_hbm)

        @pl.when(cid == 1)
        def _():
            pltpu.sync_copy(acc, deg1_hbm)


_hist_kernel = pl.kernel(
    _hist_body,
    out_type=(jax.ShapeDtypeStruct((NBIN,), jnp.float32),
              jax.ShapeDtypeStruct((NBIN,), jnp.float32)),
    mesh=_SC_MESH,
    scratch_types=[
        pltpu.VMEM_SHARED((NBIN,), jnp.float32),
        pltpu.VMEM((KH, CH), jnp.int32),
        pltpu.VMEM((CH,), jnp.float32),
        pltpu.SemaphoreType.DMA,
    ],
)


def _agg_body(y_hbm, src_hbm, dst_hbm, zeros_hbm, outa_hbm, outb_hbm,
              acc, s0, s1, d0, d1, s0s, d0s, rows0, rows1, rows_s,
              sem0, sem1, sems0, sems1, semd0, semd1):
    cid = lax.axis_index("c")
    sid = lax.axis_index("s")

    def init(base, cnt):
        @pl.when(cid == 0)
        def _():
            pltpu.sync_copy(y_hbm.at[pl.ds(base, cnt)],
                            acc.at[pl.ds(base, cnt)])

        @pl.when(cid == 1)
        def _():
            pltpu.sync_copy(zeros_hbm.at[pl.ds(base, cnt)],
                            acc.at[pl.ds(base, cnt)])

    _row_split(sid, init)
    plsc.subcore_barrier()

    def run(start, cnt):
        def iload(k, sbuf, dbuf, sems, semd):
            pltpu.async_copy(src_hbm.at[pl.ds((start + k) * CH, CH)],
                             sbuf, sems)
            pltpu.async_copy(dst_hbm.at[pl.ds((start + k) * CH, CH)],
                             dbuf, semd)

        def iwait(k, sbuf, dbuf, sems, semd):
            pltpu.make_async_copy(src_hbm.at[pl.ds((start + k) * CH, CH)],
                                  sbuf, sems).wait()
            pltpu.make_async_copy(dst_hbm.at[pl.ds((start + k) * CH, CH)],
                                  dbuf, semd).wait()

        def gather(sbuf, buf, sem):
            pltpu.async_copy(y_hbm.at[sbuf], buf, sem)

        def gwait(sbuf, buf, sem):
            pltpu.make_async_copy(y_hbm.at[sbuf], buf, sem).wait()

        iload(0, s0, d0, sems0, semd0)
        iwait(0, s0, d0, sems0, semd0)
        gather(s0, rows0, sem0)

        @pl.when(1 < cnt)
        def _():
            iload(1, s1, d1, sems1, semd1)
            iwait(1, s1, d1, sems1, semd1)
            gather(s1, rows1, sem1)

        def pair(j, carry):
            k0 = 2 * j
            gwait(s0, rows0, sem0)
            pltpu.sync_copy(rows0, acc.at[d0], add=True)

            @pl.when(k0 + 2 < cnt)
            def _():
                iload(k0 + 2, s0, d0, sems0, semd0)
                iwait(k0 + 2, s0, d0, sems0, semd0)
                gather(s0, rows0, sem0)

            gwait(s1, rows1, sem1)
            pltpu.sync_copy(rows1, acc.at[d1], add=True)

            @pl.when(k0 + 3 < cnt)
            def _():
                iload(k0 + 3, s1, d1, sems1, semd1)
                iwait(k0 + 3, s1, d1, sems1, semd1)
                gather(s1, rows1, sem1)

            return carry

        lax.fori_loop(0, cnt // 2, pair, 0)
        if cnt % 2:
            gwait(s0, rows0, sem0)
            pltpu.sync_copy(rows0, acc.at[d0], add=True)

    def run_serial(start_edge, cnt):
        def chunk(k, carry):
            off = start_edge + k * CHS
            pltpu.sync_copy(src_hbm.at[pl.ds(off, CHS)], s0s)
            pltpu.sync_copy(dst_hbm.at[pl.ds(off, CHS)], d0s)
            pltpu.async_copy(y_hbm.at[s0s], rows_s, sem0).wait()
            pltpu.sync_copy(rows_s, acc.at[d0s], add=True)
            return carry

        lax.fori_loop(0, cnt, chunk, 0)

    @pl.when(cid == 0)
    def _():
        run(sid * KAF, KAF)

    @pl.when(cid == 1)
    def _():
        run_serial(BA + sid * (KAS * CHS), KAS)

    plsc.subcore_barrier()

    def flush(base, cnt):
        @pl.when(cid == 0)
        def _():
            pltpu.sync_copy(acc.at[pl.ds(base, cnt)],
                            outa_hbm.at[pl.ds(base, cnt)])

        @pl.when(cid == 1)
        def _():
            pltpu.sync_copy(acc.at[pl.ds(base, cnt)],
                            outb_hbm.at[pl.ds(base, cnt)])

    _row_split(sid, flush)


_agg_kernel = pl.kernel(
    _agg_body,
    out_type=(jax.ShapeDtypeStruct((N, D), jnp.float32),
              jax.ShapeDtypeStruct((N, D), jnp.float32)),
    mesh=_SC_MESH,
    scratch_types=[
        pltpu.VMEM_SHARED((NBIN, D), jnp.float32),
        pltpu.VMEM((CH,), jnp.int32),
        pltpu.VMEM((CH,), jnp.int32),
        pltpu.VMEM((CH,), jnp.int32),
        pltpu.VMEM((CH,), jnp.int32),
        pltpu.VMEM((CHS,), jnp.int32),
        pltpu.VMEM((CHS,), jnp.int32),
        pltpu.VMEM((CH, D), jnp.float32),
        pltpu.VMEM((CH, D), jnp.float32),
        pltpu.VMEM((CHS, D), jnp.float32),
        pltpu.SemaphoreType.DMA,
        pltpu.SemaphoreType.DMA,
        pltpu.SemaphoreType.DMA,
        pltpu.SemaphoreType.DMA,
        pltpu.SemaphoreType.DMA,
        pltpu.SemaphoreType.DMA,
    ],
)


def _dec_body(h_hbm, ein_hbm, eout_hbm, embi_hbm, embo_hbm,
              ia0, ia1, ib0, ib1, ias, ibs, ra0, ra1, rb0, rb1, ras, rbs,
              sa0, sa1, sb0, sb1, sia0, sia1, sib0, sib1):
    cid = lax.axis_index("c")
    sid = lax.axis_index("s")

    def run(start, cnt):
        def iload(k, ba, bb, sema, semb):
            pltpu.async_copy(ein_hbm.at[pl.ds((start + k) * CH, CH)],
                             ba, sema)
            pltpu.async_copy(eout_hbm.at[pl.ds((start + k) * CH, CH)],
                             bb, semb)

        def iwait(k, ba, bb, sema, semb):
            pltpu.make_async_copy(ein_hbm.at[pl.ds((start + k) * CH, CH)],
                                  ba, sema).wait()
            pltpu.make_async_copy(eout_hbm.at[pl.ds((start + k) * CH, CH)],
                                  bb, semb).wait()

        def gather(ba, bb, bufa, bufb, sema, semb):
            pltpu.async_copy(h_hbm.at[ba], bufa, sema)
            pltpu.async_copy(h_hbm.at[bb], bufb, semb)

        def gwait(ba, bb, bufa, bufb, sema, semb):
            pltpu.make_async_copy(h_hbm.at[ba], bufa, sema).wait()
            pltpu.make_async_copy(h_hbm.at[bb], bufb, semb).wait()

        def emit(bufa, bufb, k):
            base = (start + k) * CH
            pltpu.sync_copy(bufa, embi_hbm.at[pl.ds(base, CH)])
            pltpu.sync_copy(bufb, embo_hbm.at[pl.ds(base, CH)])

        iload(0, ia0, ib0, sia0, sib0)
        iwait(0, ia0, ib0, sia0, sib0)
        gather(ia0, ib0, ra0, rb0, sa0, sb0)

        @pl.when(1 < cnt)
        def _():
            iload(1, ia1, ib1, sia1, sib1)
            iwait(1, ia1, ib1, sia1, sib1)
            gather(ia1, ib1, ra1, rb1, sa1, sb1)

        def pair(j, carry):
            k0 = 2 * j
            gwait(ia0, ib0, ra0, rb0, sa0, sb0)
            emit(ra0, rb0, k0)

            @pl.when(k0 + 2 < cnt)
            def _():
                iload(k0 + 2, ia0, ib0, sia0, sib0)
                iwait(k0 + 2, ia0, ib0, sia0, sib0)
                gather(ia0, ib0, ra0, rb0, sa0, sb0)

            gwait(ia1, ib1, ra1, rb1, sa1, sb1)
            emit(ra1, rb1, k0 + 1)

            @pl.when(k0 + 3 < cnt)
            def _():
                iload(k0 + 3, ia1, ib1, sia1, sib1)
                iwait(k0 + 3, ia1, ib1, sia1, sib1)
                gather(ia1, ib1, ra1, rb1, sa1, sb1)

            return carry

        lax.fori_loop(0, cnt // 2, pair, 0)
        if cnt % 2:
            gwait(ia0, ib0, ra0, rb0, sa0, sb0)
            emit(ra0, rb0, cnt - 1)

    def run_serial(start_pair, cnt):
        def chunk(k, carry):
            off = start_pair + k * CHS
            pltpu.sync_copy(ein_hbm.at[pl.ds(off, CHS)], ias)
            pltpu.sync_copy(eout_hbm.at[pl.ds(off, CHS)], ibs)
            cpa = pltpu.async_copy(h_hbm.at[ias], ras, sa0)
            cpb = pltpu.async_copy(h_hbm.at[ibs], rbs, sb0)
            cpa.wait()
            cpb.wait()
            pltpu.sync_copy(ras, embi_hbm.at[pl.ds(off, CHS)])
            pltpu.sync_copy(rbs, embo_hbm.at[pl.ds(off, CHS)])
            return carry

        lax.fori_loop(0, cnt, chunk, 0)

    @pl.when(cid == 0)
    def _():
        run(sid * KDF, KDF)

    if KDS:
        @pl.when(cid == 1)
        def _():
            run_serial(BD + sid * (KDS * CHS), KDS)


_dec_kernel = pl.kernel(
    _dec_body,
    out_type=(jax.ShapeDtypeStruct((EL2, D), jnp.float32),
              jax.ShapeDtypeStruct((EL2, D), jnp.float32)),
    mesh=_SC_MESH,
    scratch_types=[
        pltpu.VMEM((CH,), jnp.int32),
        pltpu.VMEM((CH,), jnp.int32),
        pltpu.VMEM((CH,), jnp.int32),
        pltpu.VMEM((CH,), jnp.int32),
        pltpu.VMEM((CHS,), jnp.int32),
        pltpu.VMEM((CHS,), jnp.int32),
        pltpu.VMEM((CH, D), jnp.float32),
        pltpu.VMEM((CH, D), jnp.float32),
        pltpu.VMEM((CH, D), jnp.float32),
        pltpu.VMEM((CH, D), jnp.float32),
        pltpu.VMEM((CHS, D), jnp.float32),
        pltpu.VMEM((CHS, D), jnp.float32),
        pltpu.SemaphoreType.DMA,
        pltpu.SemaphoreType.DMA,
        pltpu.SemaphoreType.DMA,
        pltpu.SemaphoreType.DMA,
        pltpu.SemaphoreType.DMA,
        pltpu.SemaphoreType.DMA,
        pltpu.SemaphoreType.DMA,
        pltpu.SemaphoreType.DMA,
    ],
)


BLK = 1000


def _dinv(da_ref, db_ref):
    return lax.rsqrt(da_ref[...] + db_ref[...] + 1.0)


def _tc1_call(x, W1, dega, degb):
    def body(x_ref, w_ref, da_ref, db_ref, y_ref):
        xw = jnp.dot(x_ref[...], w_ref[...],
                     preferred_element_type=jnp.float32)
        y_ref[...] = xw * _dinv(da_ref, db_ref)

    return pl.pallas_call(
        body,
        grid=(N // BLK,),
        in_specs=[pl.BlockSpec((BLK, D), lambda i: (i, 0)),
                  pl.BlockSpec((D, D), lambda i: (0, 0)),
                  pl.BlockSpec((BLK, 1), lambda i: (i, 0)),
                  pl.BlockSpec((BLK, 1), lambda i: (i, 0))],
        out_specs=pl.BlockSpec((BLK, D), lambda i: (i, 0)),
        out_shape=jax.ShapeDtypeStruct((N, D), jnp.float32),
    )(x, W1, dega, degb)


def _tc2_call(a0, a1, dega, degb, b1, W2):
    def body(a0_ref, a1_ref, da_ref, db_ref, b_ref, w_ref, y_ref):
        dinv = _dinv(da_ref, db_ref)
        h = jnp.maximum(dinv * (a0_ref[...] + a1_ref[...]) + b_ref[...], 0.0)
        y_ref[...] = jnp.dot(h, w_ref[...],
                             preferred_element_type=jnp.float32) * dinv

    return pl.pallas_call(
        body,
        grid=(N // BLK,),
        in_specs=[pl.BlockSpec((BLK, D), lambda i: (i, 0)),
                  pl.BlockSpec((BLK, D), lambda i: (i, 0)),
                  pl.BlockSpec((BLK, 1), lambda i: (i, 0)),
                  pl.BlockSpec((BLK, 1), lambda i: (i, 0)),
                  pl.BlockSpec((D,), lambda i: (0,)),
                  pl.BlockSpec((D, D), lambda i: (0, 0))],
        out_specs=pl.BlockSpec((BLK, D), lambda i: (i, 0)),
        out_shape=jax.ShapeDtypeStruct((N, D), jnp.float32),
    )(a0, a1, dega, degb, b1, W2)


def _tc3_call(c0, c1, dega, degb, b2):
    def body(c0_ref, c1_ref, da_ref, db_ref, b_ref, h_ref):
        h_ref[...] = (_dinv(da_ref, db_ref) * (c0_ref[...] + c1_ref[...])
                      + b_ref[...])

    return pl.pallas_call(
        body,
        grid=(N // BLK,),
        in_specs=[pl.BlockSpec((BLK, D), lambda i: (i, 0)),
                  pl.BlockSpec((BLK, D), lambda i: (i, 0)),
                  pl.BlockSpec((BLK, 1), lambda i: (i, 0)),
                  pl.BlockSpec((BLK, 1), lambda i: (i, 0)),
                  pl.BlockSpec((D,), lambda i: (0,))],
        out_specs=pl.BlockSpec((BLK, D), lambda i: (i, 0)),
        out_shape=jax.ShapeDtypeStruct((N, D), jnp.float32),
    )(c0, c1, dega, degb, b2)


DBLK = 4096


def _tc4_call(embi, embo):
    def body(a_ref, b_ref, p_ref):
        d = a_ref[...] - b_ref[...]
        ones = jnp.ones((D, 1), jnp.float32)
        sq = jnp.dot(d * d, ones, preferred_element_type=jnp.float32)
        p_ref[...] = 1.0 / (jnp.exp((sq + 1e-12 - R_DEC) / T_DEC) + 1.0)

    return pl.pallas_call(
        body,
        grid=((EL + DBLK - 1) // DBLK,),
        in_specs=[pl.BlockSpec((DBLK, D), lambda i: (i, 0)),
                  pl.BlockSpec((DBLK, D), lambda i: (i, 0))],
        out_specs=pl.BlockSpec((DBLK, 1), lambda i: (i, 0)),
        out_shape=jax.ShapeDtypeStruct((EL, 1), jnp.float32),
    )(embi, embo)


def kernel(node_features, edge_index, edge_label_index, W1, b1, W2, b2):
    x = node_features.astype(jnp.float32)
    srcp = jnp.concatenate(
        [edge_index[0], jnp.zeros((E2 - E,), jnp.int32)])
    dstp = jnp.concatenate(
        [edge_index[1], jnp.full((E2 - E,), N, jnp.int32)])
    dstp2 = dstp.reshape(NCHE, CH)
    einp = jnp.concatenate(
        [edge_label_index[0], jnp.zeros((EL2 - EL,), jnp.int32)])
    eoutp = jnp.concatenate(
        [edge_label_index[1], jnp.zeros((EL2 - EL,), jnp.int32)])
    zeros_nd = jnp.zeros((N, D), jnp.float32)
    zeros_n = jnp.zeros((NBIN,), jnp.float32)

    deg0, deg1 = _hist_kernel(dstp2, zeros_n)
    dega = deg0[:N].reshape(N, 1)
    degb = deg1[:N].reshape(N, 1)

    y1 = _tc1_call(x, W1, dega, degb)
    a0, a1 = _agg_kernel(y1, srcp, dstp, zeros_nd)
    y2 = _tc2_call(a0, a1, dega, degb, b1, W2)
    c0, c1 = _agg_kernel(y2, srcp, dstp, zeros_nd)
    h2 = _tc3_call(c0, c1, dega, degb, b2)

    embi, embo = _dec_kernel(h2, einp, eoutp)
    return _tc4_call(embi, embo).reshape(EL)
